# Initial kernel scaffold; baseline (speedup 1.0000x reference)
import jax
import jax.numpy as jnp
from jax.experimental import pallas as pl


def kernel(input_tokens, token_embedding):
    raise NotImplementedError("write your pallas kernel here")


# speedup vs baseline: 12.8621x; 12.8621x over previous
import math
import functools

import jax
import jax.numpy as jnp
from jax import lax
from jax.experimental import pallas as pl
from jax.experimental.pallas import tpu as pltpu
from jax.experimental.pallas import tpu_sc as plsc

_INFO = plsc.get_sparse_core_info()
_NC = _INFO.num_cores
_NS = _INFO.num_subcores
_NW = _NC * _NS
_LANES = _INFO.num_lanes


def _pe_table(max_length, d):
    position = jnp.arange(max_length, dtype=jnp.float32)[:, None]
    div_term = jnp.exp(
        jnp.arange(0, d, 2, dtype=jnp.float32) * (-math.log(10000.0) / d))
    pe = jnp.zeros((max_length, d), dtype=jnp.float32)
    pe = pe.at[:, 0::2].set(jnp.sin(position * div_term))
    pe = pe.at[:, 1::2].set(jnp.cos(position * div_term))
    return pe


def _build_fused_table(token_embedding, pe, scale):
    V, D = token_embedding.shape
    L = pe.shape[0]

    def body(emb_ref, pe_ref, out_ref):
        out_ref[...] = (emb_ref[...][:, None, :] * scale
                        + pe_ref[...][None, :, :])

    return pl.pallas_call(
        body,
        out_shape=jax.ShapeDtypeStruct((V, L, D), jnp.float32),
    )(token_embedding, pe)


def _sc_lookup(input_tokens, fused, B, L):
    R, D = fused.shape
    seq_per_w = B // _NW
    rows_per_w = seq_per_w * L
    chunk = 128
    n_chunks = rows_per_w // chunk
    n_groups = seq_per_w // _LANES

    mesh = plsc.VectorSubcoreMesh(core_axis_name="c", subcore_axis_name="s")

    @functools.partial(
        pl.kernel,
        out_type=jax.ShapeDtypeStruct((B * L, D), jnp.float32),
        mesh=mesh,
        compiler_params=pltpu.CompilerParams(needs_layout_passes=False),
        scratch_types=[
            pltpu.VMEM((seq_per_w * L,), jnp.int32),
            pltpu.VMEM((rows_per_w,), jnp.int32),
            pltpu.VMEM((chunk, D), jnp.float32),
            pltpu.VMEM((chunk, D), jnp.float32),
            pltpu.SemaphoreType.DMA,
            pltpu.SemaphoreType.DMA,
        ],
    )
    def sc_kernel(tok_hbm, fused_hbm, out_hbm, tok_v, idx_v, buf0, buf1,
                  gsem0, gsem1):
        wid = lax.axis_index("s") * _NC + lax.axis_index("c")
        base_seq = wid * seq_per_w
        pltpu.sync_copy(
            tok_hbm.at[pl.ds(base_seq * L, seq_per_w * L)], tok_v)

        lane = lax.broadcasted_iota(jnp.int32, (_LANES,), 0)
        zeros = jnp.zeros((_LANES,), jnp.int32)

        for g in range(n_groups):
            base16 = (lane + g * _LANES) * L
            p0 = plsc.load_gather(tok_v, [base16])
            plsc.store_scatter(idx_v, [base16], p0 * L)

            def step(j, carry):
                p, nc, ev = carry
                c = plsc.load_gather(tok_v, [base16 + j])
                nc = jnp.where((c >= 36) & (c <= 41), 2, nc)
                change_lt12 = (p >= 12) | (nc > 0)
                change = jnp.where(c < 12, change_lt12, p < 12)
                nc = jnp.where(c < 12, nc - 1, nc)
                ev = ev + change.astype(jnp.int32)
                plsc.store_scatter(idx_v, [base16 + j], c * L + ev)
                return c, nc, ev

            lax.fori_loop(1, L, step, (p0, zeros, zeros))

        out_base = wid * rows_per_w
        bufs = (buf0, buf1)
        sems = (gsem0, gsem1)
        copy0 = pltpu.async_copy(
            fused_hbm.at[idx_v.at[pl.ds(0, chunk)]], bufs[0], sems[0])
        prev = copy0
        for k in range(1, n_chunks):
            b = k & 1
            cur = pltpu.async_copy(
                fused_hbm.at[idx_v.at[pl.ds(k * chunk, chunk)]],
                bufs[b], sems[b])
            prev.wait()
            pltpu.sync_copy(bufs[1 - b],
                            out_hbm.at[pl.ds(out_base + (k - 1) * chunk,
                                             chunk)])
            prev = cur
        prev.wait()
        b = (n_chunks - 1) & 1
        pltpu.sync_copy(bufs[b],
                        out_hbm.at[pl.ds(out_base + (n_chunks - 1) * chunk,
                                         chunk)])

    return sc_kernel(input_tokens.reshape(B * L), fused)


def kernel(input_tokens, token_embedding):
    B, L = input_tokens.shape
    V, D = token_embedding.shape
    pe = _pe_table(L, D)
    fused = _build_fused_table(token_embedding, pe, math.sqrt(D))
    out = _sc_lookup(input_tokens, fused.reshape(V * L, D), B, L)
    return out.reshape(B, L, D)
